# speedup vs baseline: 3.3899x; 3.3899x over previous
import functools

import jax
import jax.numpy as jnp
from jax import lax
from jax.experimental import pallas as pl
from jax.experimental.pallas import tpu as pltpu
from jax.experimental.pallas import tpu_sc as plsc

NC = 2
NS = 16
NW = NC * NS

B = 4096 * 50
D = 128
BPW = B // NW
CHUNK = 400
NCHUNK = BPW // CHUNK

_mesh = plsc.VectorSubcoreMesh(
    core_axis_name="c", subcore_axis_name="s", num_cores=NC, num_subcores=NS
)


@functools.partial(
    pl.kernel,
    out_type=(
        jax.ShapeDtypeStruct((B, D), jnp.float32),
        jax.ShapeDtypeStruct((B, D), jnp.float32),
    ),
    mesh=_mesh,
    scratch_types=[
        pltpu.VMEM((CHUNK,), jnp.int32),
        pltpu.VMEM((CHUNK, D), jnp.float32),
        pltpu.SemaphoreType.DMA,
    ],
)
def _embed_lookup(src_idx, tgt_idx, src_table, tgt_table,
                  src_out, tgt_out, idx_v, rows_v, sem):
    wid = lax.axis_index("s") * NC + lax.axis_index("c")
    base = wid * BPW

    def one_table(idx_hbm, table_hbm, out_hbm):
        @pl.loop(0, NCHUNK)
        def _(i):
            off = base + i * CHUNK
            pltpu.sync_copy(idx_hbm.at[pl.ds(off, CHUNK)], idx_v)
            pltpu.async_copy(table_hbm.at[idx_v], rows_v, sem).wait()
            pltpu.sync_copy(rows_v, out_hbm.at[pl.ds(off, CHUNK)])

    one_table(src_idx, src_table, src_out)
    one_table(tgt_idx, tgt_table, tgt_out)


def kernel(src_indices, tgt_indices, src_table, tgt_table):
    shape = src_indices.shape
    src_flat = src_indices.reshape(-1).astype(jnp.int32)
    tgt_flat = tgt_indices.reshape(-1).astype(jnp.int32)
    src_out, tgt_out = _embed_lookup(src_flat, tgt_flat, src_table, tgt_table)
    return (src_out.reshape(*shape, D), tgt_out.reshape(*shape, D))


# speedup vs baseline: 3.5358x; 1.0430x over previous
import functools

import jax
import jax.numpy as jnp
from jax import lax
from jax.experimental import pallas as pl
from jax.experimental.pallas import tpu as pltpu
from jax.experimental.pallas import tpu_sc as plsc

NC = 2
NS = 16
NW = NC * NS

B = 4096 * 50
D = 128
BPW = B // NW
NBUF = 4
CHUNK = 200
NCHUNK = BPW // CHUNK

_mesh = plsc.VectorSubcoreMesh(
    core_axis_name="c", subcore_axis_name="s", num_cores=NC, num_subcores=NS
)


@functools.partial(
    pl.kernel,
    out_type=(
        jax.ShapeDtypeStruct((B, D), jnp.float32),
        jax.ShapeDtypeStruct((B, D), jnp.float32),
    ),
    mesh=_mesh,
    scratch_types=[
        pltpu.VMEM((BPW,), jnp.int32),
        pltpu.VMEM((BPW,), jnp.int32),
        [pltpu.VMEM((CHUNK, D), jnp.float32) for _ in range(NBUF)],
        [pltpu.SemaphoreType.DMA for _ in range(NBUF)],
        [pltpu.SemaphoreType.DMA for _ in range(NBUF)],
    ],
)
def _embed_lookup(src_idx, tgt_idx, src_table, tgt_table,
                  src_out, tgt_out, sidx_v, tidx_v, rows, gsem, wsem):
    wid = lax.axis_index("s") * NC + lax.axis_index("c")
    base = wid * BPW

    pltpu.sync_copy(src_idx.at[pl.ds(base, BPW)], sidx_v)
    pltpu.sync_copy(tgt_idx.at[pl.ds(base, BPW)], tidx_v)

    def one_table(idx_v, table_hbm, out_hbm):
        def gather(j, b):
            return pltpu.make_async_copy(
                table_hbm.at[idx_v.at[pl.ds(j * CHUNK, CHUNK)]], rows[b], gsem[b])

        def writeback(j, b):
            return pltpu.make_async_copy(
                rows[b], out_hbm.at[pl.ds(base + j * CHUNK, CHUNK)], wsem[b])

        for b in range(NBUF):
            gather(b, b).start()

        @pl.loop(0, NCHUNK - NBUF, step=NBUF)
        def _(i):
            for b in range(NBUF):
                gather(i + b, b).wait()
                writeback(i + b, b).start()
            for b in range(NBUF):
                writeback(i + b, b).wait()
                gather(i + NBUF + b, b).start()

        last = NCHUNK - NBUF
        for b in range(NBUF):
            gather(last + b, b).wait()
            writeback(last + b, b).start()
        for b in range(NBUF):
            writeback(last + b, b).wait()

    one_table(sidx_v, src_table, src_out)
    one_table(tidx_v, tgt_table, tgt_out)


def kernel(src_indices, tgt_indices, src_table, tgt_table):
    shape = src_indices.shape
    src_flat = src_indices.reshape(-1).astype(jnp.int32)
    tgt_flat = tgt_indices.reshape(-1).astype(jnp.int32)
    src_out, tgt_out = _embed_lookup(src_flat, tgt_flat, src_table, tgt_table)
    return (src_out.reshape(*shape, D), tgt_out.reshape(*shape, D))


# speedup vs baseline: 5.8763x; 1.6619x over previous
import functools

import jax
import jax.numpy as jnp
from jax import lax
from jax.experimental import pallas as pl
from jax.experimental.pallas import tpu as pltpu
from jax.experimental.pallas import tpu_sc as plsc

NC = 2
NS = 16
NW = NC * NS

B = 4096 * 50
D = 128
BPW = B // NW
NBUF = 4
CHUNK = 200
NCHUNK = BPW // CHUNK

_mesh = plsc.VectorSubcoreMesh(
    core_axis_name="c", subcore_axis_name="s", num_cores=NC, num_subcores=NS
)


SENT = 4096
SLEN = 50
SPW = SENT // NW
SPC = CHUNK // SLEN


@functools.partial(
    pl.kernel,
    out_type=(
        jax.ShapeDtypeStruct((SENT, SLEN, D), jnp.float32),
        jax.ShapeDtypeStruct((SENT, SLEN, D), jnp.float32),
    ),
    mesh=_mesh,
    scratch_types=[
        pltpu.VMEM((BPW,), jnp.int32),
        pltpu.VMEM((BPW,), jnp.int32),
        [pltpu.VMEM((CHUNK, D), jnp.float32) for _ in range(NBUF)],
        [pltpu.SemaphoreType.DMA for _ in range(NBUF)],
        [pltpu.SemaphoreType.DMA for _ in range(NBUF)],
    ],
)
def _embed_lookup(src_idx, tgt_idx, src_table, tgt_table,
                  src_out, tgt_out, sidx_v, tidx_v, rows, gsem, wsem):
    wid = lax.axis_index("s") * NC + lax.axis_index("c")
    base = wid * BPW
    sbase = wid * SPW

    pltpu.sync_copy(src_idx.at[pl.ds(base, BPW)], sidx_v)
    pltpu.sync_copy(tgt_idx.at[pl.ds(base, BPW)], tidx_v)

    def one_table(idx_v, table_hbm, out_hbm):
        def gather(j, b):
            return pltpu.make_async_copy(
                table_hbm.at[idx_v.at[pl.ds(j * CHUNK, CHUNK)]], rows[b], gsem[b])

        def writeback_piece(j, b, k):
            return pltpu.make_async_copy(
                rows[b].at[pl.ds(k * SLEN, SLEN)],
                out_hbm.at[sbase + j * SPC + k], wsem[b])

        def writeback_start(j, b):
            for k in range(SPC):
                writeback_piece(j, b, k).start()

        def writeback_wait(j, b):
            for k in range(SPC):
                writeback_piece(j, b, k).wait()

        for b in range(NBUF):
            gather(b, b).start()

        @pl.loop(0, NCHUNK - NBUF, step=NBUF)
        def _(i):
            for b in range(NBUF):
                gather(i + b, b).wait()
                writeback_start(i + b, b)
            for b in range(NBUF):
                writeback_wait(i + b, b)
                gather(i + NBUF + b, b).start()

        last = NCHUNK - NBUF
        for b in range(NBUF):
            gather(last + b, b).wait()
            writeback_start(last + b, b)
        for b in range(NBUF):
            writeback_wait(last + b, b)

    one_table(sidx_v, src_table, src_out)
    one_table(tidx_v, tgt_table, tgt_out)


def kernel(src_indices, tgt_indices, src_table, tgt_table):
    src_flat = src_indices.reshape(-1).astype(jnp.int32)
    tgt_flat = tgt_indices.reshape(-1).astype(jnp.int32)
    return _embed_lookup(src_flat, tgt_flat, src_table, tgt_table)


# speedup vs baseline: 5.8960x; 1.0034x over previous
import functools

import jax
import jax.numpy as jnp
from jax import lax
from jax.experimental import pallas as pl
from jax.experimental.pallas import tpu as pltpu
from jax.experimental.pallas import tpu_sc as plsc

NC = 2
NS = 16
NW = NC * NS

B = 4096 * 50
D = 128
BPW = B // NW
NBUF = 4
CHUNK = 200
NCHUNK = BPW // CHUNK

_mesh = plsc.VectorSubcoreMesh(
    core_axis_name="c", subcore_axis_name="s", num_cores=NC, num_subcores=NS
)


SENT = 4096
SLEN = 50
SPW = SENT // NW
SPC = CHUNK // SLEN


@functools.partial(
    pl.kernel,
    out_type=(
        jax.ShapeDtypeStruct((SENT, SLEN, D), jnp.float32),
        jax.ShapeDtypeStruct((SENT, SLEN, D), jnp.float32),
    ),
    mesh=_mesh,
    compiler_params=pltpu.CompilerParams(use_tc_tiling_on_sc=True),
    scratch_types=[
        pltpu.VMEM((BPW,), jnp.int32),
        pltpu.VMEM((BPW,), jnp.int32),
        [pltpu.VMEM((CHUNK, D), jnp.float32) for _ in range(NBUF)],
        [pltpu.SemaphoreType.DMA for _ in range(NBUF)],
        [pltpu.SemaphoreType.DMA for _ in range(NBUF)],
    ],
)
def _embed_lookup(src_idx, tgt_idx, src_table, tgt_table,
                  src_out, tgt_out, sidx_v, tidx_v, rows, gsem, wsem):
    wid = lax.axis_index("s") * NC + lax.axis_index("c")
    base = wid * BPW
    sbase = wid * SPW

    pltpu.sync_copy(src_idx.at[pl.ds(base, BPW)], sidx_v)
    pltpu.sync_copy(tgt_idx.at[pl.ds(base, BPW)], tidx_v)

    def one_table(idx_v, table_hbm, out_hbm):
        def gather(j, b):
            return pltpu.make_async_copy(
                table_hbm.at[idx_v.at[pl.ds(j * CHUNK, CHUNK)]], rows[b], gsem[b])

        def writeback_piece(j, b, k):
            return pltpu.make_async_copy(
                rows[b].at[pl.ds(k * SLEN, SLEN)],
                out_hbm.at[sbase + j * SPC + k], wsem[b])

        def writeback_start(j, b):
            for k in range(SPC):
                writeback_piece(j, b, k).start()

        def writeback_wait(j, b):
            for k in range(SPC):
                writeback_piece(j, b, k).wait()

        for b in range(NBUF):
            gather(b, b).start()

        @pl.loop(0, NCHUNK - NBUF, step=NBUF)
        def _(i):
            for b in range(NBUF):
                gather(i + b, b).wait()
                writeback_start(i + b, b)
            for b in range(NBUF):
                writeback_wait(i + b, b)
                gather(i + NBUF + b, b).start()

        last = NCHUNK - NBUF
        for b in range(NBUF):
            gather(last + b, b).wait()
            writeback_start(last + b, b)
        for b in range(NBUF):
            writeback_wait(last + b, b)

    one_table(sidx_v, src_table, src_out)
    one_table(tidx_v, tgt_table, tgt_out)


def kernel(src_indices, tgt_indices, src_table, tgt_table):
    src_flat = src_indices.reshape(-1).astype(jnp.int32)
    tgt_flat = tgt_indices.reshape(-1).astype(jnp.int32)
    return _embed_lookup(src_flat, tgt_flat, src_table, tgt_table)


# speedup vs baseline: 5.9773x; 1.0138x over previous
import functools

import jax
import jax.numpy as jnp
from jax import lax
from jax.experimental import pallas as pl
from jax.experimental.pallas import tpu as pltpu
from jax.experimental.pallas import tpu_sc as plsc

NC = 2
NS = 16
NW = NC * NS

B = 4096 * 50
D = 128
BPW = B // NW
NBUF = 4
CHUNK = 200
NCHUNK = BPW // CHUNK

SENT = 4096
SLEN = 50
SPW = SENT // NW
SPC = CHUNK // SLEN

_mesh = plsc.VectorSubcoreMesh(
    core_axis_name="c", subcore_axis_name="s", num_cores=NC, num_subcores=NS
)


@functools.partial(
    pl.kernel,
    out_type=jax.ShapeDtypeStruct((SENT, SLEN, D), jnp.float32),
    mesh=_mesh,
    scratch_types=[
        pltpu.VMEM((BPW,), jnp.int32),
        [pltpu.VMEM((CHUNK, D), jnp.float32) for _ in range(NBUF)],
        [pltpu.SemaphoreType.DMA for _ in range(NBUF)],
        [pltpu.SemaphoreType.DMA for _ in range(NBUF)],
    ],
)
def _embed_lookup(idx_hbm, table_hbm, out_hbm, idx_v, rows, gsem, wsem):
    wid = lax.axis_index("s") * NC + lax.axis_index("c")
    base = wid * BPW
    sbase = wid * SPW

    pltpu.sync_copy(idx_hbm.at[pl.ds(base, BPW)], idx_v)

    def gather(j, b):
        return pltpu.make_async_copy(
            table_hbm.at[idx_v.at[pl.ds(j * CHUNK, CHUNK)]], rows[b], gsem[b])

    def writeback_piece(j, b, k):
        return pltpu.make_async_copy(
            rows[b].at[pl.ds(k * SLEN, SLEN)],
            out_hbm.at[sbase + j * SPC + k], wsem[b])

    def writeback_start(j, b):
        for k in range(SPC):
            writeback_piece(j, b, k).start()

    def writeback_wait(j, b):
        for k in range(SPC):
            writeback_piece(j, b, k).wait()

    for b in range(NBUF):
        gather(b, b).start()

    @pl.loop(0, NCHUNK - NBUF, step=NBUF)
    def _(i):
        for b in range(NBUF):
            gather(i + b, b).wait()
            writeback_start(i + b, b)
        for b in range(NBUF):
            writeback_wait(i + b, b)
            gather(i + NBUF + b, b).start()

    last = NCHUNK - NBUF
    for b in range(NBUF):
        gather(last + b, b).wait()
        writeback_start(last + b, b)
    for b in range(NBUF):
        writeback_wait(last + b, b)


def kernel(src_indices, tgt_indices, src_table, tgt_table):
    src_flat = src_indices.reshape(-1).astype(jnp.int32)
    tgt_flat = tgt_indices.reshape(-1).astype(jnp.int32)
    src_out = _embed_lookup(src_flat, src_table)
    tgt_out = _embed_lookup(tgt_flat, tgt_table)
    return (src_out, tgt_out)


# speedup vs baseline: 10.6887x; 1.7882x over previous
import functools

import jax
import jax.numpy as jnp
from jax import lax
from jax.experimental import pallas as pl
from jax.experimental.pallas import tpu as pltpu
from jax.experimental.pallas import tpu_sc as plsc

NC = 2
NS = 16
NW = NC * NS

SENT = 4096
SLEN = 50
B = SENT * SLEN
D = 128
BPW = B // NW
NBUF = 4
CHUNK = 200
NCHUNK = BPW // CHUNK

_mesh = plsc.VectorSubcoreMesh(
    core_axis_name="c", subcore_axis_name="s", num_cores=NC, num_subcores=NS
)


@functools.partial(
    pl.kernel,
    out_type=(
        jax.ShapeDtypeStruct((B, D), jnp.float32),
        jax.ShapeDtypeStruct((B, D), jnp.float32),
    ),
    mesh=_mesh,
    scratch_types=[
        pltpu.VMEM((BPW,), jnp.int32),
        pltpu.VMEM((BPW,), jnp.int32),
        [pltpu.VMEM((CHUNK, D), jnp.float32) for _ in range(NBUF)],
        [pltpu.SemaphoreType.DMA for _ in range(NBUF)],
        [pltpu.SemaphoreType.DMA for _ in range(NBUF)],
    ],
)
def _embed_lookup(src_idx, tgt_idx, src_table, tgt_table,
                  src_out, tgt_out, sidx_v, tidx_v, rows, gsem, wsem):
    wid = lax.axis_index("s") * NC + lax.axis_index("c")
    base = wid * BPW

    pltpu.sync_copy(src_idx.at[pl.ds(base, BPW)], sidx_v)
    pltpu.sync_copy(tgt_idx.at[pl.ds(base, BPW)], tidx_v)

    def one_table(idx_v, table_hbm, out_hbm):
        def gather(j, b):
            return pltpu.make_async_copy(
                table_hbm.at[idx_v.at[pl.ds(j * CHUNK, CHUNK)]], rows[b], gsem[b])

        def writeback(j, b):
            return pltpu.make_async_copy(
                rows[b], out_hbm.at[pl.ds(base + j * CHUNK, CHUNK)], wsem[b])

        for b in range(NBUF):
            gather(b, b).start()

        @pl.loop(0, NCHUNK - NBUF, step=NBUF)
        def _(i):
            for b in range(NBUF):
                gather(i + b, b).wait()
                writeback(i + b, b).start()
            for b in range(NBUF):
                writeback(i + b, b).wait()
                gather(i + NBUF + b, b).start()

        last = NCHUNK - NBUF
        for b in range(NBUF):
            gather(last + b, b).wait()
            writeback(last + b, b).start()
        for b in range(NBUF):
            writeback(last + b, b).wait()

    one_table(sidx_v, src_table, src_out)
    one_table(tidx_v, tgt_table, tgt_out)


def kernel(src_indices, tgt_indices, src_table, tgt_table):
    src_t = src_indices.astype(jnp.int32).T.reshape(-1)
    tgt_t = tgt_indices.astype(jnp.int32).T.reshape(-1)
    src_out, tgt_out = _embed_lookup(src_t, tgt_t, src_table, tgt_table)
    src_emb = src_out.reshape(SLEN, SENT, D).transpose(1, 0, 2)
    tgt_emb = tgt_out.reshape(SLEN, SENT, D).transpose(1, 0, 2)
    return (src_emb, tgt_emb)
